# baseline (speedup 1.0000x reference)
import functools

import jax
import jax.numpy as jnp
from jax import lax
from jax.experimental import pallas as pl
from jax.experimental.pallas import tpu as pltpu
from jax.experimental.pallas import tpu_sc as plsc

N, C, H, W = 4, 256, 56, 56
G, Cg, Kg = 8, 32, 9
Hp, Wp = H + 2, W + 2
HT, WT = Hp + 2, Wp + 2
HW = H * W
R = N * HW * G
J = Kg * 4
OM_DIM = G * Kg * 3
OMC = G * Kg * 4

NC_SC, NS_SC = 2, 16
NTILE = NC_SC * NS_SC
PT = R // NTILE
PC = 56
NCH = PT // PC
NBLK = NTILE * NCH


def _vproj_body(x_ref, w_ref, b_ref, o_ref):
    t = lax.dot_general(
        x_ref[0].reshape(C, HW), w_ref[...], (((0,), (0,)), ((), ())),
        preferred_element_type=jnp.float32) + b_ref[...]
    t = t.astype(jnp.bfloat16).reshape(H, W, C)
    o_ref[0] = jnp.pad(t, ((2, 2), (2, 2), (0, 0)))


def _vproj_tbl(x_cm, wt, b):
    return pl.pallas_call(
        _vproj_body,
        grid=(N,),
        in_specs=[
            pl.BlockSpec((1, C, H, W), lambda n: (n, 0, 0, 0)),
            pl.BlockSpec((C, C), lambda n: (0, 0)),
            pl.BlockSpec((1, C), lambda n: (0, 0)),
        ],
        out_specs=pl.BlockSpec((1, HT, WT, C), lambda n: (n, 0, 0, 0)),
        out_shape=jax.ShapeDtypeStruct((N, HT, WT, C), jnp.bfloat16),
    )(x_cm, wt, b.reshape(1, C))


def _oproj_body(x_ref, w_ref, b_ref, o_ref):
    t = jnp.dot(x_ref[0], w_ref[...], preferred_element_type=jnp.float32) \
        + b_ref[...]
    o_ref[0] = t.T.reshape(C, H, W)


def _oproj(x, wt, b):
    return pl.pallas_call(
        _oproj_body,
        grid=(N,),
        in_specs=[
            pl.BlockSpec((1, HW, C), lambda n: (n, 0, 0)),
            pl.BlockSpec((C, C), lambda n: (0, 0)),
            pl.BlockSpec((1, C), lambda n: (0, 0)),
        ],
        out_specs=pl.BlockSpec((1, C, H, W), lambda n: (n, 0, 0, 0)),
        out_shape=jax.ShapeDtypeStruct((N, C, H, W), jnp.float32),
    )(x, wt, b.reshape(1, C))


def _offsets_body(y_ref, dwk_ref, dwb_ref, wx_ref, wy_ref, wm_ref,
                  bx_ref, by_ref, bm_ref, idx_out, w_out):
    n = pl.program_id(0)
    yt = y_ref[0].reshape(C, HW).T.reshape(H, W, C)
    ypad = jnp.pad(yt, ((1, 1), (1, 1), (0, 0)))
    acc = dwb_ref[...].reshape(1, 1, C)
    dw = jnp.zeros((H, W, C), jnp.float32) + acc
    for t in range(9):
        dy, dx = t // 3, t % 3
        dw = dw + ypad[dy:dy + H, dx:dx + W, :] * dwk_ref[t].reshape(1, 1, C)
    dw2 = dw.reshape(HW, C)
    offx = jnp.dot(dw2, wx_ref[...], preferred_element_type=jnp.float32) \
        + bx_ref[...]
    offy = jnp.dot(dw2, wy_ref[...], preferred_element_type=jnp.float32) \
        + by_ref[...]
    msk = jnp.dot(dw2, wm_ref[...], preferred_element_type=jnp.float32) \
        + bm_ref[...]

    row = lax.broadcasted_iota(jnp.int32, (HW, OMC), 0)
    col = lax.broadcasted_iota(jnp.int32, (HW, OMC), 1)
    wcoord = (row % W).astype(jnp.float32)
    ycoord = (row // W).astype(jnp.float32)
    g = col // 36
    k = (col % 36) // 4
    cc = col % 4
    dx_c = cc % 2
    dy_c = cc // 2
    kdx = (k % 3 - 1).astype(jnp.float32)
    kdy = (k // 3 - 1).astype(jnp.float32)

    px = wcoord + 1.0 + kdx + offx
    py = ycoord + 1.0 + kdy + offy
    x0 = jnp.floor(px)
    y0 = jnp.floor(py)
    fx = px - x0
    fy = py - y0
    x0c = jnp.clip(x0, -1.0, Wp - 1.0).astype(jnp.int32)
    y0c = jnp.clip(y0, -1.0, Hp - 1.0).astype(jnp.int32)
    base = ((n * HT + (y0c + 1)) * WT + (x0c + 1)) * G + g
    idx_out[0] = base + (dy_c * WT + dx_c) * G
    sx = jnp.where(dx_c == 0, 1.0 - fx, fx)
    sy = jnp.where(dy_c == 0, 1.0 - fy, fy)
    w_out[0] = sx * sy * msk


def _offsets(y_cm, dwk, dwb, wx, wy, wm, bx, by, bm):
    ispec = [
        pl.BlockSpec((1, C, H, W), lambda n: (n, 0, 0, 0)),
        pl.BlockSpec((9, C), lambda n: (0, 0)),
        pl.BlockSpec((1, C), lambda n: (0, 0)),
        pl.BlockSpec((C, OMC), lambda n: (0, 0)),
        pl.BlockSpec((C, OMC), lambda n: (0, 0)),
        pl.BlockSpec((C, OMC), lambda n: (0, 0)),
        pl.BlockSpec((1, OMC), lambda n: (0, 0)),
        pl.BlockSpec((1, OMC), lambda n: (0, 0)),
        pl.BlockSpec((1, OMC), lambda n: (0, 0)),
    ]
    ospec = pl.BlockSpec((1, HW, OMC), lambda n: (n, 0, 0))
    return pl.pallas_call(
        _offsets_body,
        grid=(N,),
        in_specs=ispec,
        out_specs=[ospec, ospec],
        out_shape=[jax.ShapeDtypeStruct((N, HW, OMC), jnp.int32),
                   jax.ShapeDtypeStruct((N, HW, OMC), jnp.float32)],
    )(y_cm, dwk, dwb.reshape(1, C), wx, wy, wm,
      bx.reshape(1, OMC), by.reshape(1, OMC), bm.reshape(1, OMC))


def _sc_body(tbl, idxh, wh, outh, idx_v, w_v, rows_v, out_v,
             gsem0, gsem1, isem0, isem1, osem0, osem1):
    wid = lax.axis_index("s") * NC_SC + lax.axis_index("c")
    gsems = (gsem0, gsem1)
    isems = (isem0, isem1)
    osems = (osem0, osem1)

    def copy_iw(c, b):
        blk = wid * NCH + c
        pltpu.async_copy(idxh.at[blk], idx_v.at[b], isems[b])
        pltpu.async_copy(wh.at[blk], w_v.at[b], isems[b])

    def wait_iw(b):
        pltpu.make_async_copy(idxh.at[0], idx_v.at[b], isems[b]).wait()
        pltpu.make_async_copy(wh.at[0], w_v.at[b], isems[b]).wait()

    def issue_gathers(b):
        def issue(p, _):
            pltpu.async_copy(tbl.at[idx_v.at[b, p]],
                             rows_v.at[b, pl.ds(p * J, J)], gsems[b])
            return ()

        lax.fori_loop(0, PC, issue, (), unroll=False)

    def drain_gathers(b):
        pltpu.make_async_copy(tbl.at[pl.ds(0, PC * J)], rows_v.at[b],
                              gsems[b]).wait()

    def accumulate_store(c, b):
        def point(p, _):
            a0 = jnp.zeros((16,), jnp.float32)
            a1 = jnp.zeros((16,), jnp.float32)
            wv0 = w_v[b, p, pl.ds(0, 16)]
            wv1 = w_v[b, p, pl.ds(16, 16)]
            wv2 = w_v[b, p, pl.ds(20, 16)]
            for j in range(J):
                if j < 16:
                    wj = wv0[j]
                elif j < 32:
                    wj = wv1[j - 16]
                else:
                    wj = wv2[j - 20]
                va, vb = plsc.unpack(
                    rows_v[b, p * J + j, :],
                    format=plsc.PackFormat.INTERLEAVED,
                    preferred_element_type=jnp.float32)
                a0 = a0 + wj * va
                a1 = a1 + wj * vb
            out_v[b, p, pl.ds(0, 16)] = a0
            out_v[b, p, pl.ds(16, 16)] = a1
            return ()

        lax.fori_loop(0, PC, point, (), unroll=False)
        pltpu.async_copy(out_v.at[b], outh.at[pl.ds((wid * NCH + c) * PC, PC)],
                         osems[b])

    def wait_store(b):
        pltpu.make_async_copy(out_v.at[b], outh.at[pl.ds(0, PC)],
                              osems[b]).wait()

    copy_iw(0, 0)
    wait_iw(0)
    issue_gathers(0)
    copy_iw(1, 1)

    def half(c, b, cc, last):
        nb = 1 - b

        @pl.when(cc < NCH // 2 - 1 if last else cc >= 0)
        def _():
            wait_iw(nb)
            issue_gathers(nb)

        drain_gathers(b)

        @pl.when(cc >= 1)
        def _():
            wait_store(b)

        accumulate_store(c, b)

        @pl.when(cc < NCH // 2 - 1)
        def _():
            copy_iw(c + 2, b)

    def step(cc, _):
        half(2 * cc, 0, cc, False)
        half(2 * cc + 1, 1, cc, True)
        return ()

    lax.fori_loop(0, NCH // 2, step, (), unroll=False)
    wait_store(0)
    wait_store(1)


def _sc_gather(tbl_flat, idx_blk, w_blk):
    mesh = plsc.VectorSubcoreMesh(core_axis_name="c", subcore_axis_name="s",
                                  num_cores=NC_SC)
    f = pl.kernel(
        _sc_body,
        out_type=jax.ShapeDtypeStruct((R, Cg), jnp.float32),
        mesh=mesh,
        scratch_types=[
            pltpu.VMEM((2, PC, J), jnp.int32),
            pltpu.VMEM((2, PC, J), jnp.float32),
            pltpu.VMEM((2, PC * J, Cg), jnp.bfloat16),
            pltpu.VMEM((2, PC, Cg), jnp.float32),
            pltpu.SemaphoreType.DMA,
            pltpu.SemaphoreType.DMA,
            pltpu.SemaphoreType.DMA,
            pltpu.SemaphoreType.DMA,
            pltpu.SemaphoreType.DMA,
            pltpu.SemaphoreType.DMA,
        ],
        compiler_params=pltpu.CompilerParams(use_tc_tiling_on_sc=False,
                                             needs_layout_passes=False),
    )
    return f(tbl_flat, idx_blk, w_blk)


def kernel(input, y, dw_w, dw_b, om_w, om_b, vp_w, vp_b, op_w, op_b):
    lanes = jnp.arange(C)
    perm = (lanes // Cg) * Cg + (lanes % 2) * 16 + (lanes % Cg) // 2
    tbl = _vproj_tbl(input, vp_w.T[:, perm], vp_b[perm])
    tbl_flat = tbl.reshape(N * HT * WT * G, Cg)

    dwk = jnp.transpose(dw_w, (1, 2, 0)).reshape(9, C)
    colj = jnp.arange(OMC)
    gg, kk = colj // 36, (colj % 36) // 4
    rows_x = gg * 27 + 2 * kk
    wx, bx = om_w[rows_x].T, om_b[rows_x]
    wy, by = om_w[rows_x + 1].T, om_b[rows_x + 1]
    rows_m = gg * 27 + 18 + kk
    wm, bm = om_w[rows_m].T, om_b[rows_m]

    idx_out, w_out = _offsets(y, dwk, dw_b, wx, wy, wm, bx, by, bm)

    idx_blk = idx_out.reshape(NBLK, PC, J)
    w_blk = w_out.reshape(NBLK, PC, J)

    out_core = _sc_gather(tbl_flat, idx_blk, w_blk)

    return _oproj(out_core.reshape(N, HW, C), op_w.T, op_b)


# speedup vs baseline: 1.2399x; 1.2399x over previous
import functools

import jax
import jax.numpy as jnp
from jax import lax
from jax.experimental import pallas as pl
from jax.experimental.pallas import tpu as pltpu
from jax.experimental.pallas import tpu_sc as plsc

N, C, H, W = 4, 256, 56, 56
G, Cg, Kg = 8, 32, 9
Hp, Wp = H + 2, W + 2
HT, WT = Hp + 2, Wp + 2
HW = H * W
R = N * HW * G
J = Kg * 4
OM_DIM = G * Kg * 3
OMC = G * Kg * 4

NC_SC, NS_SC = 2, 16
NTILE = NC_SC * NS_SC
PT = R // NTILE
PC = 56
NCH = PT // PC
NBLK = NTILE * NCH


def _vproj_body(x_ref, w_ref, b_ref, o_ref):
    t = lax.dot_general(
        x_ref[0], w_ref[...], (((0,), (0,)), ((), ())),
        preferred_element_type=jnp.float32) + b_ref[...]
    t = t.astype(jnp.bfloat16).reshape(H, W, C)
    o_ref[0] = jnp.pad(t, ((2, 2), (2, 2), (0, 0)))


def _vproj_tbl(x_cm, wt, b):
    return pl.pallas_call(
        _vproj_body,
        grid=(N,),
        in_specs=[
            pl.BlockSpec((1, C, HW), lambda n: (n, 0, 0)),
            pl.BlockSpec((C, C), lambda n: (0, 0)),
            pl.BlockSpec((1, C), lambda n: (0, 0)),
        ],
        out_specs=pl.BlockSpec((1, HT, WT, C), lambda n: (n, 0, 0, 0)),
        out_shape=jax.ShapeDtypeStruct((N, HT, WT, C), jnp.bfloat16),
    )(x_cm, wt, b.reshape(1, C))


def _oproj_body(x_ref, w_ref, b_ref, o_ref):
    t = jnp.dot(x_ref[0], w_ref[...], preferred_element_type=jnp.float32) \
        + b_ref[...]
    o_ref[0] = t.T


def _oproj(x, wt, b):
    return pl.pallas_call(
        _oproj_body,
        grid=(N,),
        in_specs=[
            pl.BlockSpec((1, HW, C), lambda n: (n, 0, 0)),
            pl.BlockSpec((C, C), lambda n: (0, 0)),
            pl.BlockSpec((1, C), lambda n: (0, 0)),
        ],
        out_specs=pl.BlockSpec((1, C, HW), lambda n: (n, 0, 0)),
        out_shape=jax.ShapeDtypeStruct((N, C, HW), jnp.float32),
    )(x, wt, b.reshape(1, C))


def _offsets_body(y_ref, dwk_ref, dwb_ref, wx_ref, wy_ref, wm_ref,
                  bx_ref, by_ref, bm_ref, idx_out, w_out):
    n = pl.program_id(0)
    yt = y_ref[0].T.reshape(H, W, C)
    ypad = jnp.pad(yt, ((1, 1), (1, 1), (0, 0)))
    acc = dwb_ref[...].reshape(1, 1, C)
    dw = jnp.zeros((H, W, C), jnp.float32) + acc
    for t in range(9):
        dy, dx = t // 3, t % 3
        dw = dw + ypad[dy:dy + H, dx:dx + W, :] * dwk_ref[t].reshape(1, 1, C)
    dw2 = dw.reshape(HW, C)
    offx = jnp.dot(dw2, wx_ref[...], preferred_element_type=jnp.float32) \
        + bx_ref[...]
    offy = jnp.dot(dw2, wy_ref[...], preferred_element_type=jnp.float32) \
        + by_ref[...]
    msk = jnp.dot(dw2, wm_ref[...], preferred_element_type=jnp.float32) \
        + bm_ref[...]

    row = lax.broadcasted_iota(jnp.int32, (HW, OMC), 0)
    col = lax.broadcasted_iota(jnp.int32, (HW, OMC), 1)
    wcoord = (row % W).astype(jnp.float32)
    ycoord = (row // W).astype(jnp.float32)
    g = col // 36
    k = (col % 36) // 4
    cc = col % 4
    dx_c = cc % 2
    dy_c = cc // 2
    kdx = (k % 3 - 1).astype(jnp.float32)
    kdy = (k // 3 - 1).astype(jnp.float32)

    px = wcoord + 1.0 + kdx + offx
    py = ycoord + 1.0 + kdy + offy
    x0 = jnp.floor(px)
    y0 = jnp.floor(py)
    fx = px - x0
    fy = py - y0
    x0c = jnp.clip(x0, -1.0, Wp - 1.0).astype(jnp.int32)
    y0c = jnp.clip(y0, -1.0, Hp - 1.0).astype(jnp.int32)
    base = ((n * HT + (y0c + 1)) * WT + (x0c + 1)) * G + g
    idx_out[0] = base + (dy_c * WT + dx_c) * G
    sx = jnp.where(dx_c == 0, 1.0 - fx, fx)
    sy = jnp.where(dy_c == 0, 1.0 - fy, fy)
    w_out[0] = sx * sy * msk


def _offsets(y_cm, dwk, dwb, wx, wy, wm, bx, by, bm):
    ispec = [
        pl.BlockSpec((1, C, HW), lambda n: (n, 0, 0)),
        pl.BlockSpec((9, C), lambda n: (0, 0)),
        pl.BlockSpec((1, C), lambda n: (0, 0)),
        pl.BlockSpec((C, OMC), lambda n: (0, 0)),
        pl.BlockSpec((C, OMC), lambda n: (0, 0)),
        pl.BlockSpec((C, OMC), lambda n: (0, 0)),
        pl.BlockSpec((1, OMC), lambda n: (0, 0)),
        pl.BlockSpec((1, OMC), lambda n: (0, 0)),
        pl.BlockSpec((1, OMC), lambda n: (0, 0)),
    ]
    ospec = pl.BlockSpec((1, HW, OMC), lambda n: (n, 0, 0))
    return pl.pallas_call(
        _offsets_body,
        grid=(N,),
        in_specs=ispec,
        out_specs=[ospec, ospec],
        out_shape=[jax.ShapeDtypeStruct((N, HW, OMC), jnp.int32),
                   jax.ShapeDtypeStruct((N, HW, OMC), jnp.float32)],
    )(y_cm, dwk, dwb.reshape(1, C), wx, wy, wm,
      bx.reshape(1, OMC), by.reshape(1, OMC), bm.reshape(1, OMC))


def _sc_body(tbl, idxh, wh, outh, idx_v, w_v, rows_v, out_v,
             gsem0, gsem1, isem0, isem1, osem0, osem1):
    wid = lax.axis_index("s") * NC_SC + lax.axis_index("c")
    gsems = (gsem0, gsem1)
    isems = (isem0, isem1)
    osems = (osem0, osem1)

    def copy_iw(c, b):
        blk = wid * NCH + c
        pltpu.async_copy(idxh.at[blk], idx_v.at[b], isems[b])
        pltpu.async_copy(wh.at[blk], w_v.at[b], isems[b])

    def wait_iw(b):
        pltpu.make_async_copy(idxh.at[0], idx_v.at[b], isems[b]).wait()
        pltpu.make_async_copy(wh.at[0], w_v.at[b], isems[b]).wait()

    def issue_gathers(b):
        def issue(q, _):
            pltpu.async_copy(tbl.at[idx_v.at[b, q]],
                             rows_v.at[b, pl.ds(q * 2 * J, 2 * J)], gsems[b])
            return ()

        lax.fori_loop(0, PC // 2, issue, (), unroll=False)

    def drain_gathers(b):
        pltpu.make_async_copy(tbl.at[pl.ds(0, PC * J)], rows_v.at[b],
                              gsems[b]).wait()

    def accumulate_store(c, b):
        @plsc.parallel_loop(0, PC, 1, unroll=2)
        def point(p):
            a0 = jnp.zeros((16,), jnp.float32)
            a1 = jnp.zeros((16,), jnp.float32)
            wv0 = w_v[b, p, pl.ds(0, 16)]
            wv1 = w_v[b, p, pl.ds(16, 16)]
            wv2 = w_v[b, p, pl.ds(20, 16)]
            for j in range(J):
                if j < 16:
                    wj = wv0[j]
                elif j < 32:
                    wj = wv1[j - 16]
                else:
                    wj = wv2[j - 20]
                va, vb = plsc.unpack(
                    rows_v[b, p * J + j, :],
                    format=plsc.PackFormat.INTERLEAVED,
                    preferred_element_type=jnp.float32)
                a0 = a0 + wj * va
                a1 = a1 + wj * vb
            out_v[b, p, pl.ds(0, 16)] = a0
            out_v[b, p, pl.ds(16, 16)] = a1

        pltpu.async_copy(out_v.at[b], outh.at[pl.ds((wid * NCH + c) * PC, PC)],
                         osems[b])

    def wait_store(b):
        pltpu.make_async_copy(out_v.at[b], outh.at[pl.ds(0, PC)],
                              osems[b]).wait()

    copy_iw(0, 0)
    wait_iw(0)
    issue_gathers(0)
    copy_iw(1, 1)

    def half(c, b, cc, last):
        nb = 1 - b

        @pl.when(cc < NCH // 2 - 1 if last else cc >= 0)
        def _():
            wait_iw(nb)
            issue_gathers(nb)

        drain_gathers(b)

        @pl.when(cc >= 1)
        def _():
            wait_store(b)

        accumulate_store(c, b)

        @pl.when(cc < NCH // 2 - 1)
        def _():
            copy_iw(c + 2, b)

    def step(cc, _):
        half(2 * cc, 0, cc, False)
        half(2 * cc + 1, 1, cc, True)
        return ()

    lax.fori_loop(0, NCH // 2, step, (), unroll=False)
    wait_store(0)
    wait_store(1)


def _sc_gather(tbl_flat, idx_blk, w_blk):
    mesh = plsc.VectorSubcoreMesh(core_axis_name="c", subcore_axis_name="s",
                                  num_cores=NC_SC)
    f = pl.kernel(
        _sc_body,
        out_type=jax.ShapeDtypeStruct((R, Cg), jnp.float32),
        mesh=mesh,
        scratch_types=[
            pltpu.VMEM((2, PC // 2, 2 * J), jnp.int32),
            pltpu.VMEM((2, PC, J), jnp.float32),
            pltpu.VMEM((2, PC * J, Cg), jnp.bfloat16),
            pltpu.VMEM((2, PC, Cg), jnp.float32),
            pltpu.SemaphoreType.DMA,
            pltpu.SemaphoreType.DMA,
            pltpu.SemaphoreType.DMA,
            pltpu.SemaphoreType.DMA,
            pltpu.SemaphoreType.DMA,
            pltpu.SemaphoreType.DMA,
        ],
        compiler_params=pltpu.CompilerParams(use_tc_tiling_on_sc=False,
                                             needs_layout_passes=False),
    )
    return f(tbl_flat, idx_blk, w_blk)


def kernel(input, y, dw_w, dw_b, om_w, om_b, vp_w, vp_b, op_w, op_b):
    lanes = jnp.arange(C)
    perm = (lanes // Cg) * Cg + (lanes % 2) * 16 + (lanes % Cg) // 2
    tbl = _vproj_tbl(input.reshape(N, C, HW), vp_w.T[:, perm], vp_b[perm])
    tbl_flat = tbl.reshape(N * HT * WT * G, Cg)

    dwk = jnp.transpose(dw_w, (1, 2, 0)).reshape(9, C)
    colj = jnp.arange(OMC)
    gg, kk = colj // 36, (colj % 36) // 4
    rows_x = gg * 27 + 2 * kk
    wx, bx = om_w[rows_x].T, om_b[rows_x]
    wy, by = om_w[rows_x + 1].T, om_b[rows_x + 1]
    rows_m = gg * 27 + 18 + kk
    wm, bm = om_w[rows_m].T, om_b[rows_m]

    idx_out, w_out = _offsets(y.reshape(N, C, HW), dwk, dw_b,
                              wx, wy, wm, bx, by, bm)

    idx_blk = idx_out.reshape(NBLK, PC, J)
    w_blk = w_out.reshape(NBLK, PC, J)

    out_core = _sc_gather(tbl_flat, idx_blk.reshape(NBLK, PC // 2, 2 * J),
                          w_blk)

    xo = _oproj(out_core.reshape(N, HW, C), op_w.T, op_b)
    return xo.reshape(N, C, H, W)


# speedup vs baseline: 1.3658x; 1.1015x over previous
import functools

import jax
import jax.numpy as jnp
from jax import lax
from jax.experimental import pallas as pl
from jax.experimental.pallas import tpu as pltpu
from jax.experimental.pallas import tpu_sc as plsc

N, C, H, W = 4, 256, 56, 56
G, Cg, Kg = 8, 32, 9
Hp, Wp = H + 2, W + 2
HT, WT = Hp + 2, Wp + 2
HW = H * W
R = N * HW * G
J = Kg * 4
OM_DIM = G * Kg * 3
OMC = G * Kg * 4

NC_SC, NS_SC = 2, 16
NTILE = NC_SC * NS_SC
PT = R // NTILE
PC = 56
NCH = PT // PC
NBLK = NTILE * NCH


def _vproj_body(x_ref, w_ref, b_ref, o_ref):
    t = lax.dot_general(
        x_ref[0], w_ref[...], (((0,), (0,)), ((), ())),
        preferred_element_type=jnp.float32) + b_ref[...]
    t = t.astype(jnp.bfloat16).reshape(H, W, C)
    o_ref[0] = jnp.pad(t, ((2, 2), (2, 2), (0, 0)))


def _vproj_tbl(x_cm, wt, b):
    return pl.pallas_call(
        _vproj_body,
        grid=(N,),
        in_specs=[
            pl.BlockSpec((1, C, HW), lambda n: (n, 0, 0)),
            pl.BlockSpec((C, C), lambda n: (0, 0)),
            pl.BlockSpec((1, C), lambda n: (0, 0)),
        ],
        out_specs=pl.BlockSpec((1, HT, WT, C), lambda n: (n, 0, 0, 0)),
        out_shape=jax.ShapeDtypeStruct((N, HT, WT, C), jnp.bfloat16),
    )(x_cm, wt, b.reshape(1, C))


def _oproj_body(x_ref, w_ref, b_ref, o_ref):
    t = jnp.dot(x_ref[0], w_ref[...], preferred_element_type=jnp.float32) \
        + b_ref[...]
    o_ref[0] = t.T


def _oproj(x, wt, b):
    return pl.pallas_call(
        _oproj_body,
        grid=(N,),
        in_specs=[
            pl.BlockSpec((1, HW, C), lambda n: (n, 0, 0)),
            pl.BlockSpec((C, C), lambda n: (0, 0)),
            pl.BlockSpec((1, C), lambda n: (0, 0)),
        ],
        out_specs=pl.BlockSpec((1, C, HW), lambda n: (n, 0, 0)),
        out_shape=jax.ShapeDtypeStruct((N, C, HW), jnp.float32),
    )(x, wt, b.reshape(1, C))


def _offsets_body(y_ref, dwk_ref, dwb_ref, wx_ref, wy_ref, wm_ref,
                  bx_ref, by_ref, bm_ref, idx_out, w_out):
    n = pl.program_id(0)
    yt = y_ref[0].T.reshape(H, W, C)
    ypad = jnp.pad(yt, ((1, 1), (1, 1), (0, 0)))
    acc = dwb_ref[...].reshape(1, 1, C)
    dw = jnp.zeros((H, W, C), jnp.float32) + acc
    for t in range(9):
        dy, dx = t // 3, t % 3
        dw = dw + ypad[dy:dy + H, dx:dx + W, :] * dwk_ref[t].reshape(1, 1, C)
    dw2 = dw.reshape(HW, C)
    offx = jnp.dot(dw2, wx_ref[...], preferred_element_type=jnp.float32) \
        + bx_ref[...]
    offy = jnp.dot(dw2, wy_ref[...], preferred_element_type=jnp.float32) \
        + by_ref[...]
    msk = jnp.dot(dw2, wm_ref[...], preferred_element_type=jnp.float32) \
        + bm_ref[...]

    row = lax.broadcasted_iota(jnp.int32, (HW, OMC), 0)
    col = lax.broadcasted_iota(jnp.int32, (HW, OMC), 1)
    wcoord = (row % W).astype(jnp.float32)
    ycoord = (row // W).astype(jnp.float32)
    g = col // 36
    k = (col % 36) // 4
    cc = col % 4
    dx_c = cc % 2
    dy_c = cc // 2
    kdx = (k % 3 - 1).astype(jnp.float32)
    kdy = (k // 3 - 1).astype(jnp.float32)

    px = wcoord + 1.0 + kdx + offx
    py = ycoord + 1.0 + kdy + offy
    x0 = jnp.floor(px)
    y0 = jnp.floor(py)
    fx = px - x0
    fy = py - y0
    x0c = jnp.clip(x0, -1.0, Wp - 1.0).astype(jnp.int32)
    y0c = jnp.clip(y0, -1.0, Hp - 1.0).astype(jnp.int32)
    base = ((n * HT + (y0c + 1)) * WT + (x0c + 1)) * G + g
    idx_out[0] = base + (dy_c * WT + dx_c) * G
    sx = jnp.where(dx_c == 0, 1.0 - fx, fx)
    sy = jnp.where(dy_c == 0, 1.0 - fy, fy)
    w_out[0] = sx * sy * msk


def _offsets(y_cm, dwk, dwb, wx, wy, wm, bx, by, bm):
    ispec = [
        pl.BlockSpec((1, C, HW), lambda n: (n, 0, 0)),
        pl.BlockSpec((9, C), lambda n: (0, 0)),
        pl.BlockSpec((1, C), lambda n: (0, 0)),
        pl.BlockSpec((C, OMC), lambda n: (0, 0)),
        pl.BlockSpec((C, OMC), lambda n: (0, 0)),
        pl.BlockSpec((C, OMC), lambda n: (0, 0)),
        pl.BlockSpec((1, OMC), lambda n: (0, 0)),
        pl.BlockSpec((1, OMC), lambda n: (0, 0)),
        pl.BlockSpec((1, OMC), lambda n: (0, 0)),
    ]
    ospec = pl.BlockSpec((1, HW, OMC), lambda n: (n, 0, 0))
    return pl.pallas_call(
        _offsets_body,
        grid=(N,),
        in_specs=ispec,
        out_specs=[ospec, ospec],
        out_shape=[jax.ShapeDtypeStruct((N, HW, OMC), jnp.int32),
                   jax.ShapeDtypeStruct((N, HW, OMC), jnp.float32)],
    )(y_cm, dwk, dwb.reshape(1, C), wx, wy, wm,
      bx.reshape(1, OMC), by.reshape(1, OMC), bm.reshape(1, OMC))


def _sc_body(tbl, idxh, wh, outh, idx_v, w_v, rows_v, out_v,
             gsem0, gsem1, isem0, isem1, osem0, osem1):
    wid = lax.axis_index("s") * NC_SC + lax.axis_index("c")
    gsems = (gsem0, gsem1)
    isems = (isem0, isem1)
    osems = (osem0, osem1)

    def copy_iw(c, b):
        blk = wid * NCH + c
        pltpu.async_copy(idxh.at[blk], idx_v.at[b], isems[b])
        pltpu.async_copy(wh.at[blk], w_v.at[b], isems[b])

    def wait_iw(b):
        pltpu.make_async_copy(idxh.at[0], idx_v.at[b], isems[b]).wait()
        pltpu.make_async_copy(wh.at[0], w_v.at[b], isems[b]).wait()

    def issue_gathers(b):
        @plsc.parallel_loop(0, PC // 2, 1, unroll=2)
        def issue(q):
            pltpu.async_copy(tbl.at[idx_v.at[b, q]],
                             rows_v.at[b, pl.ds(q * 2 * J, 2 * J)], gsems[b])

    def drain_gathers(b):
        pltpu.make_async_copy(tbl.at[pl.ds(0, PC * J)], rows_v.at[b],
                              gsems[b]).wait()

    def accumulate_store(c, b):
        @plsc.parallel_loop(0, PC, 1, unroll=4)
        def point(p):
            a0 = jnp.zeros((16,), jnp.float32)
            a1 = jnp.zeros((16,), jnp.float32)
            wv0 = w_v[b, p, pl.ds(0, 16)]
            wv1 = w_v[b, p, pl.ds(16, 16)]
            wv2 = w_v[b, p, pl.ds(20, 16)]
            for j in range(J):
                if j < 16:
                    wj = wv0[j]
                elif j < 32:
                    wj = wv1[j - 16]
                else:
                    wj = wv2[j - 20]
                va, vb = plsc.unpack(
                    rows_v[b, p * J + j, :],
                    format=plsc.PackFormat.INTERLEAVED,
                    preferred_element_type=jnp.float32)
                a0 = a0 + wj * va
                a1 = a1 + wj * vb
            out_v[b, p, pl.ds(0, 16)] = a0
            out_v[b, p, pl.ds(16, 16)] = a1

        pltpu.async_copy(out_v.at[b], outh.at[pl.ds((wid * NCH + c) * PC, PC)],
                         osems[b])

    def wait_store(b):
        pltpu.make_async_copy(out_v.at[b], outh.at[pl.ds(0, PC)],
                              osems[b]).wait()

    copy_iw(0, 0)
    wait_iw(0)
    issue_gathers(0)
    copy_iw(1, 1)

    def half(c, b, cc, last):
        nb = 1 - b

        @pl.when(cc < NCH // 2 - 1 if last else cc >= 0)
        def _():
            wait_iw(nb)
            issue_gathers(nb)

        drain_gathers(b)

        @pl.when(cc >= 1)
        def _():
            wait_store(b)

        accumulate_store(c, b)

        @pl.when(cc < NCH // 2 - 1)
        def _():
            copy_iw(c + 2, b)

    def step(cc, _):
        half(2 * cc, 0, cc, False)
        half(2 * cc + 1, 1, cc, True)
        return ()

    lax.fori_loop(0, NCH // 2, step, (), unroll=False)
    wait_store(0)
    wait_store(1)


def _sc_gather(tbl_flat, idx_blk, w_blk):
    mesh = plsc.VectorSubcoreMesh(core_axis_name="c", subcore_axis_name="s",
                                  num_cores=NC_SC)
    f = pl.kernel(
        _sc_body,
        out_type=jax.ShapeDtypeStruct((R, Cg), jnp.float32),
        mesh=mesh,
        scratch_types=[
            pltpu.VMEM((2, PC // 2, 2 * J), jnp.int32),
            pltpu.VMEM((2, PC, J), jnp.float32),
            pltpu.VMEM((2, PC * J, Cg), jnp.bfloat16),
            pltpu.VMEM((2, PC, Cg), jnp.float32),
            pltpu.SemaphoreType.DMA,
            pltpu.SemaphoreType.DMA,
            pltpu.SemaphoreType.DMA,
            pltpu.SemaphoreType.DMA,
            pltpu.SemaphoreType.DMA,
            pltpu.SemaphoreType.DMA,
        ],
        compiler_params=pltpu.CompilerParams(use_tc_tiling_on_sc=False,
                                             needs_layout_passes=False),
    )
    return f(tbl_flat, idx_blk, w_blk)


def kernel(input, y, dw_w, dw_b, om_w, om_b, vp_w, vp_b, op_w, op_b):
    lanes = jnp.arange(C)
    perm = (lanes // Cg) * Cg + (lanes % 2) * 16 + (lanes % Cg) // 2
    tbl = _vproj_tbl(input.reshape(N, C, HW), vp_w.T[:, perm], vp_b[perm])
    tbl_flat = tbl.reshape(N * HT * WT * G, Cg)

    dwk = jnp.transpose(dw_w, (1, 2, 0)).reshape(9, C)
    colj = jnp.arange(OMC)
    gg, kk = colj // 36, (colj % 36) // 4
    rows_x = gg * 27 + 2 * kk
    wx, bx = om_w[rows_x].T, om_b[rows_x]
    wy, by = om_w[rows_x + 1].T, om_b[rows_x + 1]
    rows_m = gg * 27 + 18 + kk
    wm, bm = om_w[rows_m].T, om_b[rows_m]

    idx_out, w_out = _offsets(y.reshape(N, C, HW), dwk, dw_b,
                              wx, wy, wm, bx, by, bm)

    idx_blk = idx_out.reshape(NBLK, PC, J)
    w_blk = w_out.reshape(NBLK, PC, J)

    out_core = _sc_gather(tbl_flat, idx_blk.reshape(NBLK, PC // 2, 2 * J),
                          w_blk)

    xo = _oproj(out_core.reshape(N, HW, C), op_w.T, op_b)
    return xo.reshape(N, C, H, W)
